# Initial kernel scaffold; baseline (speedup 1.0000x reference)
import jax
import jax.numpy as jnp
from jax.experimental import pallas as pl


def kernel(prop_types, hut_colors, hut_rotations, tree_types, plant_types, windmill_rotations, tower_rotations, tent_rotations, terrain, nonempty_property_mask, weight):
    raise NotImplementedError("write your pallas kernel here")


# speedup vs baseline: 4.6989x; 4.6989x over previous
import functools

import jax
import jax.numpy as jnp
from jax import lax
from jax.experimental import pallas as pl
from jax.experimental.pallas import tpu as pltpu
from jax.experimental.pallas import tpu_sc as plsc

B, H, W, E = 256, 25, 25, 64
P = H * W
PP = 640
NPROP = 9
OFFS = (0, 20, 30, 36, 46, 56, 62, 68, 74)
NROW = 82
RSTRIDE = 65
PLANE = 2 * E * P
PLANE_PAD = 80016

NC, NS = 2, 16
NW = NC * NS
BPW = B // NW
NG = PP // 16

_mesh = plsc.VectorSubcoreMesh(core_axis_name="c", subcore_axis_name="s")


@functools.partial(
    pl.kernel,
    out_type=jax.ShapeDtypeStruct((B, PLANE), jnp.float32),
    mesh=_mesh,
    scratch_types=[
        pltpu.VMEM((NPROP, PP), jnp.int32),
        pltpu.VMEM((8, PP), jnp.float32),
        pltpu.VMEM((NROW * RSTRIDE,), jnp.float32),
        pltpu.VMEM((PLANE_PAD,), jnp.float32),
    ],
    compiler_params=pltpu.CompilerParams(
        use_tc_tiling_on_sc=False, needs_layout_passes=False),
)
def _sc_embed(idx_hbm, mask_hbm, w_hbm, out_hbm, idx_v, mask_v, tbl_v, out_v):
    wid = lax.axis_index("s") * NC + lax.axis_index("c")

    pltpu.sync_copy(w_hbm, tbl_v)
    lane = lax.broadcasted_iota(jnp.int32, (16,), 0)

    def one_batch(t, carry):
        b = wid * BPW + t
        pltpu.sync_copy(idx_hbm.at[b], idx_v)
        pltpu.sync_copy(mask_hbm.at[b], mask_v)

        def one_group(g, carry2):
            px = pl.multiple_of(g * 16, 16)
            valid = lane < (P - px)
            rows = []
            for i in range(NPROP):
                r = (idx_v[i, pl.ds(px, 16)] + OFFS[i]) * RSTRIDE
                rows.append(r)
            ms = [mask_v[i, pl.ds(px, 16)] for i in range(8)]
            for ch in range(E):
                acc = plsc.load_gather(tbl_v, [rows[0] + ch]) * ms[0]
                for i in range(1, 8):
                    acc = acc + plsc.load_gather(tbl_v, [rows[i] + ch]) * ms[i]
                plsc.store_compressed(
                    out_v.at[pl.ds(ch * P + px, 16)], acc, mask=valid)
                ter = plsc.load_gather(tbl_v, [rows[8] + ch])
                plsc.store_compressed(
                    out_v.at[pl.ds((E + ch) * P + px, 16)], ter, mask=valid)
            return carry2

        lax.fori_loop(0, NG, one_group, 0, unroll=False)
        pltpu.sync_copy(out_v.at[pl.ds(0, PLANE)], out_hbm.at[b])
        return carry

    lax.fori_loop(0, BPW, one_batch, 0, unroll=False)


def kernel(prop_types, hut_colors, hut_rotations, tree_types, plant_types,
           windmill_rotations, tower_rotations, tent_rotations, terrain,
           nonempty_property_mask, weight):
    props = [prop_types, hut_colors, hut_rotations, tree_types, plant_types,
             windmill_rotations, tower_rotations, tent_rotations, terrain]
    idx = jnp.stack(
        [p.reshape(B, P).astype(jnp.int32) for p in props], axis=1)
    idx = jnp.pad(idx, ((0, 0), (0, 0), (0, PP - P)))
    mask = jnp.pad(
        nonempty_property_mask.reshape(B, 8, P).astype(jnp.float32),
        ((0, 0), (0, 0), (0, PP - P)))
    wpad = jnp.pad(weight.astype(jnp.float32),
                   ((0, 0), (0, RSTRIDE - E))).reshape(-1)
    out = _sc_embed(idx, mask, wpad)
    return out.reshape(B, 2 * E, H, W)


# speedup vs baseline: 6.2032x; 1.3202x over previous
import functools

import jax
import jax.numpy as jnp
from jax import lax
from jax.experimental import pallas as pl
from jax.experimental.pallas import tpu as pltpu
from jax.experimental.pallas import tpu_sc as plsc

B, H, W, E = 256, 25, 25, 64
P = H * W
PP = 640
NPROP = 9
OFFS = (0, 20, 30, 36, 46, 56, 62, 68, 74)
RSTRIDE = 65
ZROW = 82 * RSTRIDE
TSLICE = 5336
TALLOC = 5400
QC = 32
QW = QC * P
QPAD = QW + 16

NC, NS = 2, 16
NW = NC * NS
BPW = B // NW
NG = PP // 16

_mesh = plsc.VectorSubcoreMesh(core_axis_name="c", subcore_axis_name="s")


@functools.partial(
    pl.kernel,
    out_type=jax.ShapeDtypeStruct((B, 4 * QW), jnp.float32),
    mesh=_mesh,
    scratch_types=[
        pltpu.VMEM((NPROP, PP), jnp.int32),
        pltpu.VMEM((8, PP), jnp.float32),
        pltpu.VMEM((TALLOC,), jnp.float32),
        pltpu.VMEM((QPAD,), jnp.float32),
        pltpu.VMEM((QPAD,), jnp.float32),
        pltpu.SemaphoreType.DMA,
        pltpu.SemaphoreType.DMA,
    ],
    compiler_params=pltpu.CompilerParams(
        use_tc_tiling_on_sc=False, needs_layout_passes=False),
)
def _sc_embed(idx_hbm, mask_hbm, w_hbm, out_hbm,
              idx_v, mask_v, tbl_v, q0_v, q1_v, sem0, sem1):
    wid = lax.axis_index("s") * NC + lax.axis_index("c")
    bufs = (q0_v, q1_v)
    sems = (sem0, sem1)

    pltpu.sync_copy(w_hbm, tbl_v)
    lane = lax.broadcasted_iota(jnp.int32, (16,), 0)

    def drain(sem):
        pltpu.make_async_copy(
            out_hbm.at[0, pl.ds(0, QW)], q0_v.at[pl.ds(0, QW)], sem).wait()

    def one_batch(t, carry):
        b = wid * BPW + t
        pltpu.sync_copy(idx_hbm.at[b], idx_v)
        pltpu.sync_copy(mask_hbm.at[b], mask_v)

        for q in range(4):
            buf, sem = bufs[q % 2], sems[q % 2]
            if q < 2:
                @pl.when(t > 0)
                def _():
                    drain(sem)
            else:
                drain(sem)

            terr = q >= 2
            cbase = (q - 2) * QC if terr else q * QC

            def one_group(g, carry2, terr=terr, cbase=cbase, buf=buf):
                px = pl.multiple_of(g * 16, 16)
                valid = lane < (P - px)
                if terr:
                    r8 = (idx_v[8, pl.ds(px, 16)] * RSTRIDE
                          + (OFFS[8] * RSTRIDE + cbase))
                    for c in range(QC):
                        val = plsc.load_gather(tbl_v, [r8 + c if c else r8])
                        plsc.store_compressed(
                            buf.at[pl.ds(c * P + px, 16)], val, mask=valid)
                else:
                    rows = []
                    for i in range(8):
                        ri = (idx_v[i, pl.ds(px, 16)] * RSTRIDE
                              + (OFFS[i] * RSTRIDE + cbase))
                        mi = mask_v[i, pl.ds(px, 16)] > 0.0
                        rows.append(jnp.where(mi, ri, ZROW + cbase))
                    for c in range(QC):
                        g8 = [plsc.load_gather(tbl_v,
                                               [rows[i] + c if c else rows[i]])
                              for i in range(8)]
                        acc = (((g8[0] + g8[1]) + (g8[2] + g8[3]))
                               + ((g8[4] + g8[5]) + (g8[6] + g8[7])))
                        plsc.store_compressed(
                            buf.at[pl.ds(c * P + px, 16)], acc, mask=valid)
                return carry2

            lax.fori_loop(0, NG, one_group, 0, unroll=False)
            pltpu.async_copy(
                buf.at[pl.ds(0, QW)],
                out_hbm.at[b, pl.ds(q * QW, QW)], sem)
        return carry

    lax.fori_loop(0, BPW, one_batch, 0, unroll=False)
    drain(sem0)
    drain(sem1)


def kernel(prop_types, hut_colors, hut_rotations, tree_types, plant_types,
           windmill_rotations, tower_rotations, tent_rotations, terrain,
           nonempty_property_mask, weight):
    props = [prop_types, hut_colors, hut_rotations, tree_types, plant_types,
             windmill_rotations, tower_rotations, tent_rotations, terrain]
    idx = jnp.stack(
        [p.reshape(B, P).astype(jnp.int32) for p in props], axis=1)
    idx = jnp.pad(idx, ((0, 0), (0, 0), (0, PP - P)))
    mask = jnp.pad(
        nonempty_property_mask.reshape(B, 8, P).astype(jnp.float32),
        ((0, 0), (0, 0), (0, PP - P)))
    wpad = jnp.pad(weight.astype(jnp.float32),
                   ((0, 1), (0, RSTRIDE - E))).reshape(-1)
    wpad = jnp.pad(wpad, (0, TALLOC - wpad.shape[0]))
    out = _sc_embed(idx, mask, wpad)
    return out.reshape(B, 2 * E, H, W)


# speedup vs baseline: 8.0158x; 1.2922x over previous
import functools

import jax
import jax.numpy as jnp
from jax import lax
from jax.experimental import pallas as pl
from jax.experimental.pallas import tpu as pltpu
from jax.experimental.pallas import tpu_sc as plsc

B, H, W, E = 256, 25, 25, 64
P = H * W
PP = 640
NPROP = 9
OFFS = (0, 20, 30, 36, 46, 56, 62, 68, 74)
RSTRIDE = 33
ZROW = 82 * RSTRIDE
COPY = 83 * RSTRIDE
NCOPY = 4
TALLOC = 10960
QP = 16
QC = 32
QW = QC * P
QPAD = QW + 16

NC, NS = 2, 16
NW = NC * NS
BPW = B // NW
NG = PP // 16

_mesh = plsc.VectorSubcoreMesh(core_axis_name="c", subcore_axis_name="s")


@functools.partial(
    pl.kernel,
    out_type=jax.ShapeDtypeStruct((B, 4 * QW), jnp.float32),
    mesh=_mesh,
    scratch_types=[
        pltpu.VMEM((NPROP, PP), jnp.int32),
        pltpu.VMEM((8, PP), jnp.float32),
        pltpu.VMEM((TALLOC,), jnp.int32),
        pltpu.VMEM((QPAD,), jnp.float32),
        pltpu.VMEM((QPAD,), jnp.float32),
        pltpu.SemaphoreType.DMA,
        pltpu.SemaphoreType.DMA,
    ],
    compiler_params=pltpu.CompilerParams(
        use_tc_tiling_on_sc=False, needs_layout_passes=False),
)
def _sc_embed(idx_hbm, mask_hbm, w_hbm, out_hbm,
              idx_v, mask_v, tbl_v, q0_v, q1_v, sem0, sem1):
    wid = lax.axis_index("s") * NC + lax.axis_index("c")
    bufs = (q0_v, q1_v)
    sems = (sem0, sem1)

    pltpu.sync_copy(w_hbm, tbl_v)
    lane = lax.broadcasted_iota(jnp.int32, (16,), 0)
    rep = (lane % NCOPY) * COPY

    def drain(sem):
        pltpu.make_async_copy(
            out_hbm.at[0, pl.ds(0, QW)], q0_v.at[pl.ds(0, QW)], sem).wait()

    def one_batch(t, carry):
        b = wid * BPW + t
        pltpu.sync_copy(idx_hbm.at[b], idx_v)
        pltpu.sync_copy(mask_hbm.at[b], mask_v)

        for q in range(4):
            buf, sem = bufs[q % 2], sems[q % 2]
            if q < 2:
                @pl.when(t > 0)
                def _():
                    drain(sem)
            else:
                drain(sem)

            terr = q >= 2
            cbase = (q - 2) * QP if terr else q * QP

            def one_group(g, carry2, terr=terr, cbase=cbase, buf=buf):
                px = pl.multiple_of(g * 16, 16)
                valid = lane < (P - px)
                if terr:
                    r8 = (idx_v[8, pl.ds(px, 16)] * RSTRIDE
                          + (OFFS[8] * RSTRIDE + cbase)) + rep
                    for c in range(QP):
                        gw = plsc.load_gather(tbl_v, [r8 + c if c else r8])
                        lo, hi = plsc.unpack(
                            plsc.bitcast(gw, jnp.bfloat16),
                            format=plsc.PackFormat.INTERLEAVED,
                            preferred_element_type=jnp.float32)
                        plsc.store_compressed(
                            buf.at[pl.ds((2 * c) * P + px, 16)], lo,
                            mask=valid)
                        plsc.store_compressed(
                            buf.at[pl.ds((2 * c + 1) * P + px, 16)], hi,
                            mask=valid)
                else:
                    rows = []
                    for i in range(8):
                        ri = (idx_v[i, pl.ds(px, 16)] * RSTRIDE
                              + (OFFS[i] * RSTRIDE + cbase))
                        mi = mask_v[i, pl.ds(px, 16)] > 0.0
                        rows.append(jnp.where(mi, ri, ZROW + cbase) + rep)
                    for c in range(QP):
                        g8 = [plsc.bitcast(
                                  plsc.load_gather(
                                      tbl_v,
                                      [rows[i] + c if c else rows[i]]),
                                  jnp.bfloat16)
                              for i in range(8)]
                        acc = (((g8[0] + g8[1]) + (g8[2] + g8[3]))
                               + ((g8[4] + g8[5]) + (g8[6] + g8[7])))
                        lo, hi = plsc.unpack(
                            acc, format=plsc.PackFormat.INTERLEAVED,
                            preferred_element_type=jnp.float32)
                        plsc.store_compressed(
                            buf.at[pl.ds((2 * c) * P + px, 16)], lo,
                            mask=valid)
                        plsc.store_compressed(
                            buf.at[pl.ds((2 * c + 1) * P + px, 16)], hi,
                            mask=valid)
                return carry2

            lax.fori_loop(0, NG, one_group, 0, unroll=False)
            pltpu.async_copy(
                buf.at[pl.ds(0, QW)],
                out_hbm.at[b, pl.ds(q * QW, QW)], sem)
        return carry

    lax.fori_loop(0, BPW, one_batch, 0, unroll=False)
    drain(sem0)
    drain(sem1)


def kernel(prop_types, hut_colors, hut_rotations, tree_types, plant_types,
           windmill_rotations, tower_rotations, tent_rotations, terrain,
           nonempty_property_mask, weight):
    props = [prop_types, hut_colors, hut_rotations, tree_types, plant_types,
             windmill_rotations, tower_rotations, tent_rotations, terrain]
    idx = jnp.stack(
        [p.reshape(B, P).astype(jnp.int32) for p in props], axis=1)
    idx = jnp.pad(idx, ((0, 0), (0, 0), (0, PP - P)))
    mask = jnp.pad(
        nonempty_property_mask.reshape(B, 8, P).astype(jnp.float32),
        ((0, 0), (0, 0), (0, PP - P)))
    wb = weight.astype(jnp.bfloat16)
    lo16 = lax.bitcast_convert_type(wb[:, 0::2], jnp.uint16).astype(jnp.uint32)
    hi16 = lax.bitcast_convert_type(wb[:, 1::2], jnp.uint16).astype(jnp.uint32)
    wpk = (lo16 | (hi16 << 16)).astype(jnp.int32)
    wpk = jnp.pad(wpk, ((0, 1), (0, RSTRIDE - E // 2))).reshape(-1)
    wpk = jnp.tile(wpk, NCOPY)
    wpad = jnp.pad(wpk, (0, TALLOC - wpk.shape[0]))
    out = _sc_embed(idx, mask, wpad)
    return out.reshape(B, 2 * E, H, W)


# speedup vs baseline: 8.3449x; 1.0411x over previous
import functools

import jax
import jax.numpy as jnp
from jax import lax
from jax.experimental import pallas as pl
from jax.experimental.pallas import tpu as pltpu
from jax.experimental.pallas import tpu_sc as plsc

B, H, W, E = 256, 25, 25, 64
P = H * W
PP = 640
NPROP = 9
OFFS = (0, 20, 30, 36, 46, 56, 62, 68, 74)
RSTRIDE = 33
ZROW = 82 * RSTRIDE
COPY = 83 * RSTRIDE
NCOPY = 8
TALLOC = 21920
QP = 16
QC = 32
QW = QC * P
QPAD = QW + 16

NC, NS = 2, 16
NW = NC * NS
BPW = B // NW
NG = PP // 16

_mesh = plsc.VectorSubcoreMesh(core_axis_name="c", subcore_axis_name="s")


@functools.partial(
    pl.kernel,
    out_type=jax.ShapeDtypeStruct((B, 4 * QW), jnp.float32),
    mesh=_mesh,
    scratch_types=[
        pltpu.VMEM((NPROP, PP), jnp.int32),
        pltpu.VMEM((NPROP, PP), jnp.int32),
        pltpu.VMEM((8, PP), jnp.float32),
        pltpu.VMEM((8, PP), jnp.float32),
        pltpu.VMEM((TALLOC,), jnp.int32),
        pltpu.VMEM((QPAD,), jnp.float32),
        pltpu.VMEM((QPAD,), jnp.float32),
        pltpu.SemaphoreType.DMA,
        pltpu.SemaphoreType.DMA,
        pltpu.SemaphoreType.DMA,
    ],
    compiler_params=pltpu.CompilerParams(
        use_tc_tiling_on_sc=False, needs_layout_passes=False),
)
def _sc_embed(idx_hbm, mask_hbm, w_hbm, out_hbm,
              idx_v0, idx_v1, mask_v0, mask_v1, tbl_v, q0_v, q1_v,
              sem0, sem1, semi):
    wid = lax.axis_index("s") * NC + lax.axis_index("c")
    bufs = (q0_v, q1_v)
    sems = (sem0, sem1)

    pltpu.sync_copy(w_hbm, tbl_v)
    lane = lax.broadcasted_iota(jnp.int32, (16,), 0)
    rep = (lane % NCOPY) * COPY

    def drain(sem):
        pltpu.make_async_copy(
            out_hbm.at[0, pl.ds(0, QW)], q0_v.at[pl.ds(0, QW)], sem).wait()

    def fetch_inputs(b, iv, mv):
        pltpu.async_copy(idx_hbm.at[b], iv, semi)
        pltpu.async_copy(mask_hbm.at[b], mv, semi)

    def drain_inputs():
        pltpu.make_async_copy(idx_hbm.at[0], idx_v0, semi).wait()
        pltpu.make_async_copy(mask_hbm.at[0], mask_v0, semi).wait()

    fetch_inputs(wid * BPW, idx_v0, mask_v0)

    def one_batch(t, carry, idx_v=None, mask_v=None):
        b = wid * BPW + t

        for q in range(4):
            buf, sem = bufs[q % 2], sems[q % 2]
            if q < 2:
                @pl.when(t > 0)
                def _():
                    drain(sem)
            else:
                drain(sem)

            terr = q >= 2
            cbase = (q - 2) * QP if terr else q * QP

            def one_group(g, carry2, terr=terr, cbase=cbase, buf=buf):
                px = pl.multiple_of(g * 16, 16)
                valid = lane < (P - px)
                if terr:
                    r8 = (idx_v[8, pl.ds(px, 16)] * RSTRIDE
                          + (OFFS[8] * RSTRIDE + cbase)) + rep
                    for c in range(QP):
                        gw = plsc.load_gather(tbl_v, [r8 + c if c else r8])
                        lo, hi = plsc.unpack(
                            plsc.bitcast(gw, jnp.bfloat16),
                            format=plsc.PackFormat.INTERLEAVED,
                            preferred_element_type=jnp.float32)
                        plsc.store_compressed(
                            buf.at[pl.ds((2 * c) * P + px, 16)], lo,
                            mask=valid)
                        plsc.store_compressed(
                            buf.at[pl.ds((2 * c + 1) * P + px, 16)], hi,
                            mask=valid)
                else:
                    rows = []
                    for i in range(8):
                        ri = (idx_v[i, pl.ds(px, 16)] * RSTRIDE
                              + (OFFS[i] * RSTRIDE + cbase))
                        mi = mask_v[i, pl.ds(px, 16)] > 0.0
                        rows.append(jnp.where(mi, ri, ZROW + cbase) + rep)
                    for c in range(QP):
                        g8 = [plsc.bitcast(
                                  plsc.load_gather(
                                      tbl_v,
                                      [rows[i] + c if c else rows[i]]),
                                  jnp.bfloat16)
                              for i in range(8)]
                        acc = (((g8[0] + g8[1]) + (g8[2] + g8[3]))
                               + ((g8[4] + g8[5]) + (g8[6] + g8[7])))
                        lo, hi = plsc.unpack(
                            acc, format=plsc.PackFormat.INTERLEAVED,
                            preferred_element_type=jnp.float32)
                        plsc.store_compressed(
                            buf.at[pl.ds((2 * c) * P + px, 16)], lo,
                            mask=valid)
                        plsc.store_compressed(
                            buf.at[pl.ds((2 * c + 1) * P + px, 16)], hi,
                            mask=valid)
                return carry2

            lax.fori_loop(0, NG, one_group, 0, unroll=False)
            pltpu.async_copy(
                buf.at[pl.ds(0, QW)],
                out_hbm.at[b, pl.ds(q * QW, QW)], sem)
        return carry

    def one_pair(t2, carry):
        t_even = t2 * 2
        drain_inputs()
        fetch_inputs(wid * BPW + t_even + 1, idx_v1, mask_v1)
        one_batch(t_even, carry, idx_v=idx_v0, mask_v=mask_v0)
        drain_inputs()

        @pl.when(t_even + 2 < BPW)
        def _():
            fetch_inputs(wid * BPW + t_even + 2, idx_v0, mask_v0)
        one_batch(t_even + 1, carry, idx_v=idx_v1, mask_v=mask_v1)
        return carry

    lax.fori_loop(0, BPW // 2, one_pair, 0, unroll=False)
    drain(sem0)
    drain(sem1)


def kernel(prop_types, hut_colors, hut_rotations, tree_types, plant_types,
           windmill_rotations, tower_rotations, tent_rotations, terrain,
           nonempty_property_mask, weight):
    props = [prop_types, hut_colors, hut_rotations, tree_types, plant_types,
             windmill_rotations, tower_rotations, tent_rotations, terrain]
    idx = jnp.stack(
        [p.reshape(B, P).astype(jnp.int32) for p in props], axis=1)
    idx = jnp.pad(idx, ((0, 0), (0, 0), (0, PP - P)))
    mask = jnp.pad(
        nonempty_property_mask.reshape(B, 8, P).astype(jnp.float32),
        ((0, 0), (0, 0), (0, PP - P)))
    wb = weight.astype(jnp.bfloat16)
    lo16 = lax.bitcast_convert_type(wb[:, 0::2], jnp.uint16).astype(jnp.uint32)
    hi16 = lax.bitcast_convert_type(wb[:, 1::2], jnp.uint16).astype(jnp.uint32)
    wpk = (lo16 | (hi16 << 16)).astype(jnp.int32)
    wpk = jnp.pad(wpk, ((0, 1), (0, RSTRIDE - E // 2))).reshape(-1)
    wpk = jnp.tile(wpk, NCOPY)
    wpad = jnp.pad(wpk, (0, TALLOC - wpk.shape[0]))
    out = _sc_embed(idx, mask, wpad)
    return out.reshape(B, 2 * E, H, W)
